# Initial kernel scaffold; baseline (speedup 1.0000x reference)
import jax
import jax.numpy as jnp
from jax.experimental import pallas as pl


def kernel(z_d, z_m, d_sim, m_sim, diseases, mirnas, att_w1, att_b1, att_w2, mfc_w, mfc_b, dfc_w, dfc_b, hfc_w, hfc_b, bil_w):
    raise NotImplementedError("write your pallas kernel here")


# speedup vs baseline: 33.8555x; 33.8555x over previous
import functools

import jax
import jax.numpy as jnp
from jax import lax
from jax.experimental import pallas as pl
from jax.experimental.pallas import tpu as pltpu
from jax.experimental.pallas import tpu_sc as plsc

NUM_D = 383
NUM_M = 495
DIM = 128
HIDDEN = 512
N_PAIRS = 262144

PAD_D = 384
PAD_M = 496
PAD_N = PAD_D + PAD_M

NUM_CORES = 2
NUM_SUBCORES = 16
NUM_WORKERS = NUM_CORES * NUM_SUBCORES
CHUNK = N_PAIRS // NUM_WORKERS
LANES = 16


def _elu(x):
    return jnp.where(x > 0, x, jnp.exp(x) - 1.0)


def _tc_score_table(zd_ref, zm_ref, dsim_ref, msim_ref,
                    aw1_ref, ab1_ref, aw2_ref,
                    dtop_ref, dsimw_ref, db_ref,
                    mtop_ref, msimw_ref, mb_ref,
                    hw_ref, hb_ref, bil_ref, out_ref):
    aw1 = aw1_ref[...]
    ab1 = ab1_ref[...]
    aw2 = aw2_ref[...]

    def attn(z_ref, n_valid, n_pad):
        mask = lax.broadcasted_iota(jnp.int32, (n_pad, 1), 0) < n_valid
        betas = []
        for p in range(5):
            zp = z_ref[p]
            w = jnp.tanh(jnp.dot(zp, aw1, preferred_element_type=jnp.float32)
                         + ab1)
            s = jnp.sum(w * aw2, axis=1, keepdims=True)
            s = jnp.where(mask, s, 0.0)
            betas.append(jax.nn.sigmoid(jnp.sum(s) / n_valid))
        h = betas[0] * z_ref[0]
        for p in range(1, 5):
            h = h + betas[p] * z_ref[p]
        return h

    h1 = attn(zd_ref, NUM_D, PAD_D)
    h2 = attn(zm_ref, NUM_M, PAD_M)

    h_d = _elu(jnp.dot(h1, dtop_ref[...], preferred_element_type=jnp.float32)
               + jnp.dot(dsim_ref[...], dsimw_ref[...],
                         preferred_element_type=jnp.float32)
               + db_ref[...])
    h_m = _elu(jnp.dot(h2, mtop_ref[...], preferred_element_type=jnp.float32)
               + jnp.dot(msim_ref[...], msimw_ref[...],
                         preferred_element_type=jnp.float32)
               + mb_ref[...])

    h = jnp.concatenate([h_d, h_m], axis=0)
    h = _elu(jnp.dot(h, hw_ref[...], preferred_element_type=jnp.float32)
             + hb_ref[...])
    g = jnp.dot(h, bil_ref[...], preferred_element_type=jnp.float32)
    scores = lax.dot_general(g, h, (((1,), (1,)), ((), ())),
                             preferred_element_type=jnp.float32)
    out_ref[...] = jax.nn.sigmoid(scores)


def _sc_gather(sflat_hbm, d_hbm, m_hbm, out_hbm, d_v, m_v, idx_v, val_v, sem):
    wid = lax.axis_index("s") * NUM_CORES + lax.axis_index("c")
    base = wid * CHUNK
    pltpu.sync_copy(d_hbm.at[pl.ds(base, CHUNK)], d_v)
    pltpu.sync_copy(m_hbm.at[pl.ds(base, CHUNK)], m_v)

    vec_per_iter = 8
    n_iter = CHUNK // (LANES * vec_per_iter)

    def body(i, carry):
        for j in range(vec_per_iter):
            off = pl.multiple_of(i * (LANES * vec_per_iter) + j * LANES, LANES)
            d = d_v[pl.ds(off, LANES)]
            m = m_v[pl.ds(off, LANES)]
            dd = jnp.where(d >= NUM_D, d + (PAD_D - NUM_D), d)
            mm = jnp.where(m >= NUM_D, m + (PAD_D - NUM_D), m)
            idx_v[pl.ds(off, LANES)] = dd * PAD_N + mm
        return carry

    lax.fori_loop(0, n_iter, body, 0)
    pltpu.async_copy(sflat_hbm.at[idx_v], val_v, sem).wait()
    pltpu.sync_copy(val_v, out_hbm.at[pl.ds(base, CHUNK)])


def kernel(z_d, z_m, d_sim, m_sim, diseases, mirnas, att_w1, att_b1, att_w2,
           mfc_w, mfc_b, dfc_w, dfc_b, hfc_w, hfc_b, bil_w):
    f32 = jnp.float32

    zd_t = jnp.pad(jnp.transpose(z_d, (1, 0, 2)),
                   ((0, 0), (0, PAD_D - NUM_D), (0, 0)))
    zm_t = jnp.pad(jnp.transpose(z_m, (1, 0, 2)),
                   ((0, 0), (0, PAD_M - NUM_M), (0, 0)))
    dsim_p = jnp.pad(d_sim, ((0, PAD_D - NUM_D), (0, PAD_D - NUM_D)))
    msim_p = jnp.pad(m_sim, ((0, PAD_M - NUM_M), (0, PAD_M - NUM_M)))
    dtop = dfc_w[:DIM]
    dsimw = jnp.pad(dfc_w[DIM:], ((0, PAD_D - NUM_D), (0, 0)))
    mtop = mfc_w[:DIM]
    msimw = jnp.pad(mfc_w[DIM:], ((0, PAD_M - NUM_M), (0, 0)))
    ab1 = att_b1.reshape(1, HIDDEN)
    aw2 = att_w2.reshape(1, HIDDEN)
    db = dfc_b.reshape(1, DIM)
    mb = mfc_b.reshape(1, DIM)
    hb = hfc_b.reshape(1, DIM)

    table = pl.pallas_call(
        _tc_score_table,
        out_shape=jax.ShapeDtypeStruct((PAD_N, PAD_N), f32),
    )(zd_t, zm_t, dsim_p, msim_p, att_w1, ab1, aw2,
      dtop, dsimw, db, mtop, msimw, mb, hfc_w, hb, bil_w)

    sflat = table.reshape(PAD_N * PAD_N)

    mesh = plsc.VectorSubcoreMesh(core_axis_name="c", subcore_axis_name="s",
                                  num_cores=NUM_CORES,
                                  num_subcores=NUM_SUBCORES)
    scores = pl.kernel(
        _sc_gather,
        out_type=jax.ShapeDtypeStruct((N_PAIRS,), f32),
        mesh=mesh,
        scratch_types=[
            pltpu.VMEM((CHUNK,), jnp.int32),
            pltpu.VMEM((CHUNK,), jnp.int32),
            pltpu.VMEM((CHUNK,), jnp.int32),
            pltpu.VMEM((CHUNK,), f32),
            pltpu.SemaphoreType.DMA,
        ],
    )(sflat, diseases.astype(jnp.int32), mirnas.astype(jnp.int32))

    return scores.reshape(N_PAIRS, 1)


# speedup vs baseline: 35.7942x; 1.0573x over previous
import jax
import jax.numpy as jnp
from jax import lax
from jax.experimental import pallas as pl
from jax.experimental.pallas import tpu as pltpu
from jax.experimental.pallas import tpu_sc as plsc

NUM_D = 383
NUM_M = 495
NUM_N = NUM_D + NUM_M
DIM = 128
HIDDEN = 512
N_PAIRS = 262144

NUM_CORES = 2
NUM_SUBCORES = 16
NUM_WORKERS = NUM_CORES * NUM_SUBCORES
CHUNK = N_PAIRS // NUM_WORKERS
LANES = 16


def _elu(x):
    return jnp.where(x > 0, x, jnp.exp(x) - 1.0)


def _tc_score_table(zd_ref, zm_ref, dsim_ref, msim_ref,
                    aw1_ref, ab1_ref, aw2_ref,
                    dfc_ref, db_ref, mfc_ref, mb_ref,
                    hw_ref, hb_ref, bil_ref, out_ref):
    aw1 = aw1_ref[...]
    ab1 = ab1_ref[...]
    aw2 = aw2_ref[...]

    def attn(z_ref, n):
        betas = []
        for p in range(5):
            zp = z_ref[:, p, :]
            w = jnp.tanh(jnp.dot(zp, aw1, preferred_element_type=jnp.float32)
                         + ab1)
            s = jnp.dot(w, aw2, preferred_element_type=jnp.float32)
            betas.append(jax.nn.sigmoid(jnp.sum(s) / n))
        h = betas[0] * z_ref[:, 0, :]
        for p in range(1, 5):
            h = h + betas[p] * z_ref[:, p, :]
        return h

    h1 = attn(zd_ref, NUM_D)
    h2 = attn(zm_ref, NUM_M)

    h_d = _elu(jnp.dot(h1, dfc_ref[:DIM], preferred_element_type=jnp.float32)
               + jnp.dot(dsim_ref[...], dfc_ref[DIM:],
                         preferred_element_type=jnp.float32)
               + db_ref[...])
    h_m = _elu(jnp.dot(h2, mfc_ref[:DIM], preferred_element_type=jnp.float32)
               + jnp.dot(msim_ref[...], mfc_ref[DIM:],
                         preferred_element_type=jnp.float32)
               + mb_ref[...])

    h = jnp.concatenate([h_d, h_m], axis=0)
    h = _elu(jnp.dot(h, hw_ref[...], preferred_element_type=jnp.float32)
             + hb_ref[...])
    g = jnp.dot(h, bil_ref[...], preferred_element_type=jnp.float32)
    scores = lax.dot_general(g, h, (((1,), (1,)), ((), ())),
                             preferred_element_type=jnp.float32)
    out_ref[...] = jax.nn.sigmoid(scores)


def _sc_gather(sflat_hbm, d_hbm, m_hbm, out_hbm, d_v, m_v, idx_v, val_v, sem):
    wid = lax.axis_index("s") * NUM_CORES + lax.axis_index("c")
    base = wid * CHUNK
    pltpu.sync_copy(d_hbm.at[pl.ds(base, CHUNK)], d_v)
    pltpu.sync_copy(m_hbm.at[pl.ds(base, CHUNK)], m_v)

    vec_per_iter = 8
    n_iter = CHUNK // (LANES * vec_per_iter)

    def body(i, carry):
        for j in range(vec_per_iter):
            off = pl.multiple_of(i * (LANES * vec_per_iter) + j * LANES, LANES)
            d = d_v[pl.ds(off, LANES)]
            m = m_v[pl.ds(off, LANES)]
            idx_v[pl.ds(off, LANES)] = d * NUM_N + m
        return carry

    lax.fori_loop(0, n_iter, body, 0)
    pltpu.async_copy(sflat_hbm.at[idx_v], val_v, sem).wait()
    pltpu.sync_copy(val_v, out_hbm.at[pl.ds(base, CHUNK)])


def kernel(z_d, z_m, d_sim, m_sim, diseases, mirnas, att_w1, att_b1, att_w2,
           mfc_w, mfc_b, dfc_w, dfc_b, hfc_w, hfc_b, bil_w):
    f32 = jnp.float32

    table = pl.pallas_call(
        _tc_score_table,
        out_shape=jax.ShapeDtypeStruct((NUM_N, NUM_N), f32),
    )(z_d, z_m, d_sim, m_sim, att_w1, att_b1, att_w2,
      dfc_w, dfc_b, mfc_w, mfc_b, hfc_w, hfc_b, bil_w)

    sflat = table.reshape(NUM_N * NUM_N)

    mesh = plsc.VectorSubcoreMesh(core_axis_name="c", subcore_axis_name="s",
                                  num_cores=NUM_CORES,
                                  num_subcores=NUM_SUBCORES)
    scores = pl.kernel(
        _sc_gather,
        out_type=jax.ShapeDtypeStruct((N_PAIRS,), f32),
        mesh=mesh,
        scratch_types=[
            pltpu.VMEM((CHUNK,), jnp.int32),
            pltpu.VMEM((CHUNK,), jnp.int32),
            pltpu.VMEM((CHUNK,), jnp.int32),
            pltpu.VMEM((CHUNK,), f32),
            pltpu.SemaphoreType.DMA,
        ],
    )(sflat, diseases.astype(jnp.int32), mirnas.astype(jnp.int32))

    return scores.reshape(N_PAIRS, 1)


# speedup vs baseline: 49.7098x; 1.3888x over previous
import jax
import jax.numpy as jnp
from jax import lax
from jax.experimental import pallas as pl
from jax.experimental.pallas import tpu as pltpu
from jax.experimental.pallas import tpu_sc as plsc

NUM_D = 383
NUM_M = 495
NUM_N = NUM_D + NUM_M
DIM = 128
HIDDEN = 512
N_PAIRS = 262144

ROW_PAD = 880
COL_PAD = 896
RB = ROW_PAD // 8
CB = COL_PAD // 128
N_TILES = RB * CB
TABLE_LEN = N_TILES * 1024

NUM_CORES = 2
NUM_SUBCORES = 16
NUM_WORKERS = NUM_CORES * NUM_SUBCORES
CHUNK = N_PAIRS // NUM_WORKERS
LANES = 16


def _elu(x):
    return jnp.where(x > 0, x, jnp.exp(x) - 1.0)


def _tc_score_table(zd_ref, zm_ref, dsim_ref, msim_ref,
                    aw1_ref, ab1_ref, aw2_ref,
                    dfc_ref, db_ref, mfc_ref, mb_ref,
                    hw_ref, hb_ref, bil_ref, out_ref):
    aw1 = aw1_ref[...]
    ab1 = ab1_ref[...]
    aw2 = aw2_ref[...][None, :]

    def attn(z_ref, n):
        betas = []
        for p in range(5):
            zp = z_ref[p]
            w = jnp.tanh(jnp.dot(zp, aw1, preferred_element_type=jnp.float32)
                         + ab1)
            s = jnp.sum(w * aw2, axis=1, keepdims=True)
            betas.append(jax.nn.sigmoid(jnp.sum(s) / n))
        h = betas[0] * z_ref[0]
        for p in range(1, 5):
            h = h + betas[p] * z_ref[p]
        return h

    h1 = attn(zd_ref, NUM_D)
    h2 = attn(zm_ref, NUM_M)

    h_d = _elu(jnp.dot(h1, dfc_ref[:DIM], preferred_element_type=jnp.float32)
               + jnp.dot(dsim_ref[...], dfc_ref[DIM:],
                         preferred_element_type=jnp.float32)
               + db_ref[...])
    h_m = _elu(jnp.dot(h2, mfc_ref[:DIM], preferred_element_type=jnp.float32)
               + jnp.dot(msim_ref[...], mfc_ref[DIM:],
                         preferred_element_type=jnp.float32)
               + mb_ref[...])

    pad2 = jnp.zeros((ROW_PAD - NUM_N, DIM), jnp.float32)
    h = jnp.concatenate([h_d, h_m, pad2], axis=0)
    h = _elu(jnp.dot(h, hw_ref[...], preferred_element_type=jnp.float32)
             + hb_ref[...])
    g = jnp.dot(h, bil_ref[...], preferred_element_type=jnp.float32)
    scores = lax.dot_general(g, h, (((1,), (1,)), ((), ())),
                             preferred_element_type=jnp.float32)
    scores = jax.nn.sigmoid(scores)
    scores = jnp.concatenate(
        [scores, jnp.zeros((ROW_PAD, COL_PAD - ROW_PAD), jnp.float32)],
        axis=1)
    for rb in range(RB):
        for cb in range(CB):
            out_ref[rb * CB + cb] = scores[8 * rb:8 * rb + 8,
                                           128 * cb:128 * cb + 128]


def _sc_gather(sflat_hbm, d_hbm, m_hbm, out_hbm, d_v, m_v, idx_v, val_v, sem):
    wid = lax.axis_index("s") * NUM_CORES + lax.axis_index("c")
    base = wid * CHUNK
    pltpu.sync_copy(d_hbm.at[pl.ds(base, CHUNK)], d_v)
    pltpu.sync_copy(m_hbm.at[pl.ds(base, CHUNK)], m_v)

    vec_per_iter = 8
    n_iter = CHUNK // (LANES * vec_per_iter)

    def body(i, carry):
        for j in range(vec_per_iter):
            off = pl.multiple_of(i * (LANES * vec_per_iter) + j * LANES, LANES)
            r = d_v[pl.ds(off, LANES)]
            c = m_v[pl.ds(off, LANES)]
            tile = (r >> 3) * CB + (c >> 7)
            idx_v[pl.ds(off, LANES)] = (tile << 10) + ((r & 7) << 7) + (c & 127)
        return carry

    lax.fori_loop(0, n_iter, body, 0)
    pltpu.async_copy(sflat_hbm.at[idx_v], val_v, sem).wait()
    pltpu.sync_copy(val_v, out_hbm.at[pl.ds(base, CHUNK)])


def kernel(z_d, z_m, d_sim, m_sim, diseases, mirnas, att_w1, att_b1, att_w2,
           mfc_w, mfc_b, dfc_w, dfc_b, hfc_w, hfc_b, bil_w):
    f32 = jnp.float32

    zd_t = jnp.transpose(z_d, (1, 0, 2))
    zm_t = jnp.transpose(z_m, (1, 0, 2))
    aw2 = att_w2.reshape(HIDDEN)

    table = pl.pallas_call(
        _tc_score_table,
        out_shape=jax.ShapeDtypeStruct((N_TILES, 8, 128), f32),
    )(zd_t, zm_t, d_sim, m_sim, att_w1, att_b1, aw2,
      dfc_w, dfc_b, mfc_w, mfc_b, hfc_w, hfc_b, bil_w)

    sflat = table.reshape(TABLE_LEN)

    mesh = plsc.VectorSubcoreMesh(core_axis_name="c", subcore_axis_name="s",
                                  num_cores=NUM_CORES,
                                  num_subcores=NUM_SUBCORES)
    scores = pl.kernel(
        _sc_gather,
        out_type=jax.ShapeDtypeStruct((N_PAIRS,), f32),
        mesh=mesh,
        scratch_types=[
            pltpu.VMEM((CHUNK,), jnp.int32),
            pltpu.VMEM((CHUNK,), jnp.int32),
            pltpu.VMEM((CHUNK,), jnp.int32),
            pltpu.VMEM((CHUNK,), f32),
            pltpu.SemaphoreType.DMA,
        ],
    )(sflat, diseases.astype(jnp.int32), mirnas.astype(jnp.int32))

    return scores.reshape(N_PAIRS, 1)


# speedup vs baseline: 50.5282x; 1.0165x over previous
import jax
import jax.numpy as jnp
from jax import lax
from jax.experimental import pallas as pl
from jax.experimental.pallas import tpu as pltpu
from jax.experimental.pallas import tpu_sc as plsc

NUM_D = 383
NUM_M = 495
NUM_N = NUM_D + NUM_M
DIM = 128
HIDDEN = 512
N_PAIRS = 262144

ROW_PAD = 880
COL_PAD = 896
RB = ROW_PAD // 8
CB = COL_PAD // 128
N_TILES = RB * CB
TABLE_LEN = N_TILES * 1024

NUM_CORES = 2
NUM_SUBCORES = 16
NUM_WORKERS = NUM_CORES * NUM_SUBCORES
CHUNK = N_PAIRS // NUM_WORKERS
LANES = 16


def _elu(x):
    return jnp.where(x > 0, x, jnp.exp(x) - 1.0)


def _tc_score_table(zd_ref, zm_ref, dsim_ref, msim_ref,
                    aw1_ref, ab1_ref, aw2_ref,
                    dfc_ref, db_ref, mfc_ref, mb_ref,
                    hw_ref, hb_ref, bil_ref, out_ref):
    aw1 = aw1_ref[...]
    ab1 = ab1_ref[...]
    aw2 = aw2_ref[...][None, :]

    def attn(z_ref, n):
        betas = []
        for p in range(5):
            zp = z_ref[p]
            w = jnp.tanh(jnp.dot(zp, aw1, preferred_element_type=jnp.float32)
                         + ab1)
            s = jnp.sum(w * aw2, axis=1, keepdims=True)
            betas.append(jax.nn.sigmoid(jnp.sum(s) / n))
        h = betas[0] * z_ref[0]
        for p in range(1, 5):
            h = h + betas[p] * z_ref[p]
        return h

    h1 = attn(zd_ref, NUM_D)
    h2 = attn(zm_ref, NUM_M)

    h_d = _elu(jnp.dot(h1, dfc_ref[:DIM], preferred_element_type=jnp.float32)
               + jnp.dot(dsim_ref[...], dfc_ref[DIM:],
                         preferred_element_type=jnp.float32)
               + db_ref[...])
    h_m = _elu(jnp.dot(h2, mfc_ref[:DIM], preferred_element_type=jnp.float32)
               + jnp.dot(msim_ref[...], mfc_ref[DIM:],
                         preferred_element_type=jnp.float32)
               + mb_ref[...])

    pad2 = jnp.zeros((ROW_PAD - NUM_N, DIM), jnp.float32)
    h = jnp.concatenate([h_d, h_m, pad2], axis=0)
    h = _elu(jnp.dot(h, hw_ref[...], preferred_element_type=jnp.float32)
             + hb_ref[...])
    g = jnp.dot(h, bil_ref[...], preferred_element_type=jnp.float32)
    scores = lax.dot_general(g, h, (((1,), (1,)), ((), ())),
                             preferred_element_type=jnp.float32)
    scores = jax.nn.sigmoid(scores)
    scores = jnp.concatenate(
        [scores, jnp.zeros((ROW_PAD, COL_PAD - ROW_PAD), jnp.float32)],
        axis=1)
    for rb in range(RB):
        for cb in range(CB):
            out_ref[rb * CB + cb] = scores[8 * rb:8 * rb + 8,
                                           128 * cb:128 * cb + 128]


N_SUB = 8
SUB = CHUNK // N_SUB


def _sc_gather(sflat_hbm, d_hbm, m_hbm, out_hbm, d_v, m_v, idx_v, val_v,
               ld_sem, g_sem):
    wid = lax.axis_index("s") * NUM_CORES + lax.axis_index("c")
    base = wid * CHUNK
    ld_d = pltpu.async_copy(d_hbm.at[pl.ds(base, CHUNK)], d_v, ld_sem)
    ld_m = pltpu.async_copy(m_hbm.at[pl.ds(base, CHUNK)], m_v, ld_sem)
    ld_d.wait()
    ld_m.wait()

    vec_per_iter = 8
    n_iter = SUB // (LANES * vec_per_iter)

    gathers = []
    for k in range(N_SUB):
        kbase = k * SUB

        def body(i, carry, kbase=kbase):
            for j in range(vec_per_iter):
                off = pl.multiple_of(
                    kbase + i * (LANES * vec_per_iter) + j * LANES, LANES)
                r = d_v[pl.ds(off, LANES)]
                c = m_v[pl.ds(off, LANES)]
                tile = (r >> 3) * CB + (c >> 7)
                idx_v[pl.ds(off, LANES)] = ((tile << 10) + ((r & 7) << 7)
                                            + (c & 127))
            return carry

        lax.fori_loop(0, n_iter, body, 0)
        gathers.append(pltpu.async_copy(
            sflat_hbm.at[idx_v.at[pl.ds(kbase, SUB)]],
            val_v.at[pl.ds(kbase, SUB)], g_sem))
    for g in gathers:
        g.wait()
    pltpu.sync_copy(val_v, out_hbm.at[pl.ds(base, CHUNK)])


def kernel(z_d, z_m, d_sim, m_sim, diseases, mirnas, att_w1, att_b1, att_w2,
           mfc_w, mfc_b, dfc_w, dfc_b, hfc_w, hfc_b, bil_w):
    f32 = jnp.float32

    zd_t = jnp.transpose(z_d, (1, 0, 2))
    zm_t = jnp.transpose(z_m, (1, 0, 2))
    aw2 = att_w2.reshape(HIDDEN)

    table = pl.pallas_call(
        _tc_score_table,
        out_shape=jax.ShapeDtypeStruct((N_TILES, 8, 128), f32),
    )(zd_t, zm_t, d_sim, m_sim, att_w1, att_b1, aw2,
      dfc_w, dfc_b, mfc_w, mfc_b, hfc_w, hfc_b, bil_w)

    sflat = table.reshape(TABLE_LEN)

    mesh = plsc.VectorSubcoreMesh(core_axis_name="c", subcore_axis_name="s",
                                  num_cores=NUM_CORES,
                                  num_subcores=NUM_SUBCORES)
    scores = pl.kernel(
        _sc_gather,
        out_type=jax.ShapeDtypeStruct((N_PAIRS,), f32),
        mesh=mesh,
        scratch_types=[
            pltpu.VMEM((CHUNK,), jnp.int32),
            pltpu.VMEM((CHUNK,), jnp.int32),
            pltpu.VMEM((CHUNK,), jnp.int32),
            pltpu.VMEM((CHUNK,), f32),
            pltpu.SemaphoreType.DMA,
            pltpu.SemaphoreType.DMA,
        ],
    )(sflat, diseases.astype(jnp.int32), mirnas.astype(jnp.int32))

    return scores.reshape(N_PAIRS, 1)
